# baseline (speedup 1.0000x reference)
import functools

import jax
import jax.numpy as jnp
from jax import lax
from jax.experimental import pallas as pl
from jax.experimental.pallas import tpu as pltpu
from jax.experimental.pallas import tpu_sc as plsc

D = 768
LANES = 16
DCHUNKS = D // LANES
NW = 32
POS_PER_TILE = 64
CH = 8
NSLOTS = 3


def _trig_body(phase_ref, cos_ref, sin_ref):
    p = phase_ref[...]
    cos_ref[...] = jnp.cos(p)
    sin_ref[...] = jnp.sin(p)


def _compute_trig(phase_flat):
    n = phase_flat.shape[0]
    p2 = phase_flat.reshape(n // 128, 128)
    cos2, sin2 = pl.pallas_call(
        _trig_body,
        out_shape=(
            jax.ShapeDtypeStruct(p2.shape, jnp.float32),
            jax.ShapeDtypeStruct(p2.shape, jnp.float32),
        ),
    )(p2)
    return cos2.reshape(n), sin2.reshape(n)


def _make_sc_kernel(total_tokens, bsz, seq):
    mesh = plsc.VectorSubcoreMesh(core_axis_name="c", subcore_axis_name="s")
    chunks_per_batch = POS_PER_TILE // CH
    n_chunks = chunks_per_batch * bsz

    @functools.partial(
        pl.kernel,
        mesh=mesh,
        out_type=jax.ShapeDtypeStruct((2, total_tokens, D), jnp.float32),
        scratch_types=(
            [pltpu.VMEM((bsz * POS_PER_TILE,), jnp.int32)]
            + [pltpu.VMEM((bsz * POS_PER_TILE * LANES,), jnp.float32)] * 2
            + [pltpu.VMEM((POS_PER_TILE, D), jnp.float32)]
            + [pltpu.VMEM((CH, D), jnp.float32)] * (2 * NSLOTS)
            + [pltpu.SemaphoreType.DMA] * (4 * NSLOTS)
        ),
    )
    def sc_kernel(ids_hbm, cos_hbm, sin_hbm, wr_hbm, wi_hbm, pos_hbm,
                  out_hbm, ids_v, cos_v, sin_v, pos_v, *bufs_and_sems):
        bufr = bufs_and_sems[0:NSLOTS]
        bufi = bufs_and_sems[NSLOTS:2 * NSLOTS]
        semgr = bufs_and_sems[2 * NSLOTS:3 * NSLOTS]
        semgi = bufs_and_sems[3 * NSLOTS:4 * NSLOTS]
        semor = bufs_and_sems[4 * NSLOTS:5 * NSLOTS]
        semoi = bufs_and_sems[5 * NSLOTS:6 * NSLOTS]

        cid = lax.axis_index("c")
        sid = lax.axis_index("s")
        wid = sid * 2 + cid
        pos0 = wid * POS_PER_TILE

        for b in range(bsz):
            src = pl.ds(b * seq + pos0, POS_PER_TILE)
            dst = pl.ds(b * POS_PER_TILE, POS_PER_TILE)
            pltpu.sync_copy(ids_hbm.at[src], ids_v.at[dst])
            srcl = pl.ds((b * seq + pos0) * LANES, POS_PER_TILE * LANES)
            dstl = pl.ds(b * POS_PER_TILE * LANES, POS_PER_TILE * LANES)
            pltpu.sync_copy(cos_hbm.at[srcl], cos_v.at[dstl])
            pltpu.sync_copy(sin_hbm.at[srcl], sin_v.at[dstl])
        pltpu.sync_copy(pos_hbm.at[pl.ds(pos0, POS_PER_TILE)], pos_v)

        def chunk_coords(g):
            return g // chunks_per_batch, g % chunks_per_batch

        def start_gather(g):
            slot = g % NSLOTS
            b, h = chunk_coords(g)
            ic = ids_v.at[pl.ds(b * POS_PER_TILE + h * CH, CH)]
            return (
                pltpu.async_copy(wr_hbm.at[ic], bufr[slot], semgr[slot]),
                pltpu.async_copy(wi_hbm.at[ic], bufi[slot], semgi[slot]),
            )

        gath = {}
        outs = {}
        gath[0] = start_gather(0)
        if n_chunks > 1:
            gath[1] = start_gather(1)

        for g in range(n_chunks):
            slot = g % NSLOTS
            b, h = chunk_coords(g)
            tbase = b * POS_PER_TILE + h * CH
            for cp in gath.pop(g):
                cp.wait()

            br = bufr[slot]
            bi = bufi[slot]

            def tok_body(j, _, br=br, bi=bi, h=h, tbase=tbase):
                tsl = pl.ds((jnp.int32(tbase) + j) * LANES, LANES)
                c = cos_v[tsl]
                s = sin_v[tsl]
                rr = br.at[j]
                ri = bi.at[j]
                rp = pos_v.at[h * CH + j]

                def d_body(d, _):
                    sl = pl.ds(d * LANES, LANES)
                    x = rr[sl] + rp[sl]
                    y = ri[sl]
                    rr[sl] = x * c - y * s
                    ri[sl] = x * s + y * c
                    return _

                return lax.fori_loop(0, DCHUNKS, d_body, _)

            lax.fori_loop(0, CH, tok_body, None)

            tok = b * seq + pos0 + h * CH
            outs[g] = (
                pltpu.async_copy(br, out_hbm.at[0, pl.ds(tok, CH)], semor[slot]),
                pltpu.async_copy(bi, out_hbm.at[1, pl.ds(tok, CH)], semoi[slot]),
            )

            ng = g + 2
            if ng < n_chunks:
                prev = ng - NSLOTS
                if prev in outs:
                    for cp in outs.pop(prev):
                        cp.wait()
                gath[ng] = start_gather(ng)

        for g in sorted(outs):
            for cp in outs[g]:
                cp.wait()

    return sc_kernel


def kernel(input_ids, initial_phase, W_real, W_imag, pos_table):
    bsz, seq = input_ids.shape
    total = bsz * seq
    ids = input_ids.reshape(total).astype(jnp.int32)
    cos_f, sin_f = _compute_trig(initial_phase.reshape(total))
    cos_b = jnp.broadcast_to(cos_f[:, None], (total, LANES)).reshape(-1)
    sin_b = jnp.broadcast_to(sin_f[:, None], (total, LANES)).reshape(-1)
    sc = _make_sc_kernel(total, bsz, seq)
    out = sc(ids, cos_b, sin_b, W_real, W_imag, pos_table)
    return out.reshape(2, bsz, seq, D)


# speedup vs baseline: 1.3625x; 1.3625x over previous
import functools

import jax
import jax.numpy as jnp
from jax import lax
from jax.experimental import pallas as pl
from jax.experimental.pallas import tpu as pltpu
from jax.experimental.pallas import tpu_sc as plsc

D = 768
NW = 32
CH = 32
NSLOTS = 2
BT = 256


def _make_gather_kernel(total):
    mesh = plsc.VectorSubcoreMesh(core_axis_name="c", subcore_axis_name="s")
    tok_per_tile = total // NW
    n_chunks = tok_per_tile // CH

    @functools.partial(
        pl.kernel,
        mesh=mesh,
        out_type=(
            jax.ShapeDtypeStruct((total, D), jnp.float32),
            jax.ShapeDtypeStruct((total, D), jnp.float32),
        ),
        scratch_types=(
            [pltpu.VMEM((tok_per_tile,), jnp.int32)]
            + [pltpu.VMEM((CH, D), jnp.float32)] * (2 * NSLOTS)
            + [pltpu.SemaphoreType.DMA] * (4 * NSLOTS)
        ),
    )
    def gather_kernel(ids_hbm, wr_hbm, wi_hbm, outr_hbm, outi_hbm,
                      ids_v, *rest):
        bufr = rest[0:NSLOTS]
        bufi = rest[NSLOTS:2 * NSLOTS]
        semgr = rest[2 * NSLOTS:3 * NSLOTS]
        semgi = rest[3 * NSLOTS:4 * NSLOTS]
        semor = rest[4 * NSLOTS:5 * NSLOTS]
        semoi = rest[5 * NSLOTS:6 * NSLOTS]

        cid = lax.axis_index("c")
        sid = lax.axis_index("s")
        wid = sid * 2 + cid
        tok0 = wid * tok_per_tile

        pltpu.sync_copy(ids_hbm.at[pl.ds(tok0, tok_per_tile)], ids_v)

        def start_gather(g):
            slot = g % NSLOTS
            ic = ids_v.at[pl.ds(g * CH, CH)]
            return (
                pltpu.async_copy(wr_hbm.at[ic], bufr[slot], semgr[slot]),
                pltpu.async_copy(wi_hbm.at[ic], bufi[slot], semgi[slot]),
            )

        gath = {}
        outs = {}
        for g in range(min(NSLOTS, n_chunks)):
            gath[g] = start_gather(g)

        for g in range(n_chunks):
            slot = g % NSLOTS
            for cp in gath.pop(g):
                cp.wait()
            dst = pl.ds(tok0 + g * CH, CH)
            outs[g] = (
                pltpu.async_copy(bufr[slot], outr_hbm.at[dst], semor[slot]),
                pltpu.async_copy(bufi[slot], outi_hbm.at[dst], semoi[slot]),
            )
            ng = g + NSLOTS
            if ng < n_chunks:
                for cp in outs.pop(g):
                    cp.wait()
                gath[ng] = start_gather(ng)

        for g in sorted(outs):
            for cp in outs[g]:
                cp.wait()

    return gather_kernel


def _rotate_body(theta_ref, gr_ref, gi_ref, pos_ref, out_ref):
    th = theta_ref[...]
    c = jnp.cos(th)
    s = jnp.sin(th)
    x = gr_ref[...] + pos_ref[...]
    y = gi_ref[...]
    out_ref[0, 0] = x * c - y * s
    out_ref[1, 0] = x * s + y * c


def _rotate(theta_col, gr, gi, pos_table, bsz, seq):
    nb = seq // BT
    return pl.pallas_call(
        _rotate_body,
        grid=(bsz, nb),
        in_specs=[
            pl.BlockSpec((BT, 1), lambda b, i: (b * nb + i, 0)),
            pl.BlockSpec((BT, D), lambda b, i: (b * nb + i, 0)),
            pl.BlockSpec((BT, D), lambda b, i: (b * nb + i, 0)),
            pl.BlockSpec((BT, D), lambda b, i: (i, 0)),
        ],
        out_specs=pl.BlockSpec((2, 1, BT, D), lambda b, i: (0, b, i, 0)),
        out_shape=jax.ShapeDtypeStruct((2, bsz, seq, D), jnp.float32),
    )(theta_col, gr, gi, pos_table)


def kernel(input_ids, initial_phase, W_real, W_imag, pos_table):
    bsz, seq = input_ids.shape
    total = bsz * seq
    ids = input_ids.reshape(total).astype(jnp.int32)
    gr, gi = _make_gather_kernel(total)(ids, W_real, W_imag)
    theta_col = initial_phase.reshape(total, 1)
    return _rotate(theta_col, gr, gi, pos_table, bsz, seq)


# speedup vs baseline: 1.4045x; 1.0308x over previous
import functools

import jax
import jax.numpy as jnp
from jax import lax
from jax.experimental import pallas as pl
from jax.experimental.pallas import tpu as pltpu
from jax.experimental.pallas import tpu_sc as plsc

D = 768
NW = 32
CH = 32
NSLOTS = 2
BT = 256


def _make_gather_kernel(total):
    mesh = plsc.VectorSubcoreMesh(core_axis_name="c", subcore_axis_name="s")
    tok_per_tile = total // NW
    n_chunks = tok_per_tile // CH

    @functools.partial(
        pl.kernel,
        mesh=mesh,
        out_type=(
            jax.ShapeDtypeStruct((total, D), jnp.float32),
            jax.ShapeDtypeStruct((total, D), jnp.float32),
        ),
        scratch_types=(
            [pltpu.VMEM((tok_per_tile,), jnp.int32)]
            + [pltpu.VMEM((CH, D), jnp.float32)] * (2 * NSLOTS)
            + [pltpu.SemaphoreType.DMA] * (4 * NSLOTS)
        ),
    )
    def gather_kernel(ids_hbm, wr_hbm, wi_hbm, outr_hbm, outi_hbm,
                      ids_v, *rest):
        bufr = rest[0:NSLOTS]
        bufi = rest[NSLOTS:2 * NSLOTS]
        semgr = rest[2 * NSLOTS:3 * NSLOTS]
        semgi = rest[3 * NSLOTS:4 * NSLOTS]
        semor = rest[4 * NSLOTS:5 * NSLOTS]
        semoi = rest[5 * NSLOTS:6 * NSLOTS]

        cid = lax.axis_index("c")
        sid = lax.axis_index("s")
        wid = sid * 2 + cid
        tok0 = wid * tok_per_tile

        pltpu.sync_copy(ids_hbm.at[pl.ds(tok0, tok_per_tile)], ids_v)

        def start_gather(g):
            slot = g % NSLOTS
            ic = ids_v.at[pl.ds(g * CH, CH)]
            return (
                pltpu.async_copy(wr_hbm.at[ic], bufr[slot], semgr[slot]),
                pltpu.async_copy(wi_hbm.at[ic], bufi[slot], semgi[slot]),
            )

        gath = {}
        outs = {}
        for g in range(min(NSLOTS, n_chunks)):
            gath[g] = start_gather(g)

        for g in range(n_chunks):
            slot = g % NSLOTS
            for cp in gath.pop(g):
                cp.wait()
            dst = pl.ds(tok0 + g * CH, CH)
            outs[g] = (
                pltpu.async_copy(bufr[slot], outr_hbm.at[dst], semor[slot]),
                pltpu.async_copy(bufi[slot], outi_hbm.at[dst], semoi[slot]),
            )
            ng = g + NSLOTS
            if ng < n_chunks:
                for cp in outs.pop(g):
                    cp.wait()
                gath[ng] = start_gather(ng)

        for g in sorted(outs):
            for cp in outs[g]:
                cp.wait()

    return gather_kernel


def _rotate_body(theta_ref, gr_ref, gi_ref, pos_ref, out_ref):
    th = theta_ref[...]
    c = jnp.cos(th)
    s = jnp.sin(th)
    x = gr_ref[...] + pos_ref[...]
    y = gi_ref[...]
    out_ref[0, 0] = x * c - y * s
    out_ref[1, 0] = x * s + y * c


def _rotate_body_aliased(theta_ref, gr_ref, gi_ref, pos_ref, _prev_ref,
                         out_ref):
    _rotate_body(theta_ref, gr_ref, gi_ref, pos_ref, out_ref)


def _rotate_chunk(theta_col, gr, gi, pos_table, b, bsz, seq, prev):
    nb = seq // BT
    in_specs = [
        pl.BlockSpec((BT, 1), lambda i: (i, 0)),
        pl.BlockSpec((BT, D), lambda i: (i, 0)),
        pl.BlockSpec((BT, D), lambda i: (i, 0)),
        pl.BlockSpec((BT, D), lambda i: (i, 0)),
    ]
    operands = [theta_col, gr, gi, pos_table]
    body = _rotate_body
    aliases = {}
    if prev is not None:
        in_specs.append(pl.BlockSpec(memory_space=pl.ANY))
        operands.append(prev)
        body = _rotate_body_aliased
        aliases = {4: 0}
    return pl.pallas_call(
        body,
        grid=(nb,),
        in_specs=in_specs,
        out_specs=pl.BlockSpec((2, 1, BT, D), lambda i, b=b: (0, b, i, 0)),
        out_shape=jax.ShapeDtypeStruct((2, bsz, seq, D), jnp.float32),
        input_output_aliases=aliases,
    )(*operands)


def kernel(input_ids, initial_phase, W_real, W_imag, pos_table):
    bsz, seq = input_ids.shape
    ids = input_ids.astype(jnp.int32)
    theta = initial_phase
    gathered = []
    sc_gather = _make_gather_kernel(seq)
    for b in range(bsz):
        gathered.append(sc_gather(ids[b], W_real, W_imag))
    out = None
    for b in range(bsz):
        gr, gi = gathered[b]
        out = _rotate_chunk(theta[b].reshape(seq, 1), gr, gi, pos_table,
                            b, bsz, seq, out)
    return out
